# Initial kernel scaffold; baseline (speedup 1.0000x reference)
import jax
import jax.numpy as jnp
from jax.experimental import pallas as pl


def kernel(x, table):
    raise NotImplementedError("write your pallas kernel here")


# speedup vs baseline: 1.7279x; 1.7279x over previous
import functools
import math

import jax
import jax.numpy as jnp
import numpy as np
from jax import lax
from jax.experimental import pallas as pl
from jax.experimental.pallas import tpu as pltpu
from jax.experimental.pallas import tpu_sc as plsc

VOCAB = 1000
D_MODEL = 128
MAX_LEN = 512
SEQ = 200
SCALE = float(math.sqrt(D_MODEL))

CHUNK = 128


def _pe_extended() -> jnp.ndarray:
    pe = np.zeros((MAX_LEN, D_MODEL), dtype=np.float32)
    position = np.arange(0, MAX_LEN, dtype=np.float32)[:, None]
    div_term = np.exp(
        np.arange(0, D_MODEL, 2, dtype=np.float32) * (-math.log(10000.0) / D_MODEL)
    )
    pe[:, 0::2] = np.sin(position * div_term)
    pe[:, 1::2] = np.cos(position * div_term)
    pe = pe[:SEQ]
    pe_ext = np.concatenate([pe, pe[:CHUNK]], axis=0)
    return jnp.asarray(pe_ext)


def _make_sc_call(batch: int, seq: int):
    info = plsc.get_sparse_core_info()
    nc, ns = info.num_cores, info.num_subcores
    nw = nc * ns
    ntok = batch * seq
    assert ntok % (nw * CHUNK) == 0
    tok_per_w = ntok // nw
    nchunk = tok_per_w // CHUNK
    pe_rows = seq + CHUNK

    mesh = plsc.VectorSubcoreMesh(core_axis_name="c", subcore_axis_name="s")

    @functools.partial(
        pl.kernel,
        mesh=mesh,
        out_type=jax.ShapeDtypeStruct((ntok, D_MODEL), jnp.float32),
        scratch_types=[
            pltpu.VMEM((CHUNK,), jnp.int32),
            pltpu.VMEM((CHUNK, D_MODEL), jnp.float32),
            pltpu.VMEM((pe_rows, D_MODEL), jnp.float32),
            pltpu.SemaphoreType.DMA,
        ],
    )
    def sc_embed(x_hbm, table_hbm, pe_hbm, out_hbm, idx_v, rows_v, pe_v, sem):
        wid = lax.axis_index("s") * nc + lax.axis_index("c")
        pltpu.sync_copy(pe_hbm, pe_v)

        def chunk_body(c, carry):
            tok_base = wid * tok_per_w + c * CHUNK
            pltpu.sync_copy(x_hbm.at[pl.ds(tok_base, CHUNK)], idx_v)
            pltpu.async_copy(table_hbm.at[idx_v], rows_v, sem).wait()
            pe0 = lax.rem(c * CHUNK, seq)

            def tok_body(t, tcarry):
                for j in range(D_MODEL // 16):
                    sl = pl.ds(j * 16, 16)
                    rows_v[t, sl] = rows_v[t, sl] * SCALE + pe_v[pe0 + t, sl]
                return tcarry

            lax.fori_loop(0, CHUNK, tok_body, 0, unroll=False)
            pltpu.sync_copy(rows_v, out_hbm.at[pl.ds(tok_base, CHUNK)])
            return carry

        lax.fori_loop(0, nchunk, chunk_body, 0, unroll=False)

    return sc_embed


def kernel(x, table):
    batch, seq = x.shape
    x_flat = x.reshape(-1).astype(jnp.int32)
    pe_ext = _pe_extended()
    sc_embed = _make_sc_call(batch, seq)
    out = sc_embed(x_flat, table, pe_ext)
    return out.reshape(batch, seq, D_MODEL)


# speedup vs baseline: 2.4916x; 1.4420x over previous
import functools
import math

import jax
import jax.numpy as jnp
import numpy as np
from jax import lax
from jax.experimental import pallas as pl
from jax.experimental.pallas import tpu as pltpu
from jax.experimental.pallas import tpu_sc as plsc

VOCAB = 1000
D_MODEL = 128
MAX_LEN = 512
SEQ = 200
SCALE = float(math.sqrt(D_MODEL))

CHUNK = 128
NBUF = 4


def _pe_extended() -> jnp.ndarray:
    pe = np.zeros((MAX_LEN, D_MODEL), dtype=np.float32)
    position = np.arange(0, MAX_LEN, dtype=np.float32)[:, None]
    div_term = np.exp(
        np.arange(0, D_MODEL, 2, dtype=np.float32) * (-math.log(10000.0) / D_MODEL)
    )
    pe[:, 0::2] = np.sin(position * div_term)
    pe[:, 1::2] = np.cos(position * div_term)
    pe = pe[:SEQ]
    pe_ext = np.concatenate([pe, pe[:CHUNK]], axis=0)
    return jnp.asarray(pe_ext)


def _make_sc_call(batch: int, seq: int):
    info = plsc.get_sparse_core_info()
    nc, ns = info.num_cores, info.num_subcores
    nw = nc * ns
    ntok = batch * seq
    assert ntok % (nw * CHUNK) == 0
    tok_per_w = ntok // nw
    nchunk = tok_per_w // CHUNK
    pe_rows = seq + CHUNK

    mesh = plsc.VectorSubcoreMesh(core_axis_name="c", subcore_axis_name="s")

    @functools.partial(
        pl.kernel,
        mesh=mesh,
        out_type=jax.ShapeDtypeStruct((ntok, D_MODEL), jnp.float32),
        scratch_types=[
            pltpu.VMEM((NBUF, CHUNK), jnp.int32),
            pltpu.VMEM((NBUF, CHUNK, D_MODEL), jnp.float32),
            pltpu.VMEM((pe_rows, D_MODEL), jnp.float32),
        ]
        + [pltpu.SemaphoreType.DMA] * (2 * NBUF),
    )
    def sc_embed(x_hbm, table_hbm, pe_hbm, out_hbm, idx_v, rows_v, pe_v, *sems):
        gsem = sems[:NBUF]
        osem = sems[NBUF:]
        wid = lax.axis_index("s") * nc + lax.axis_index("c")
        tok0 = wid * tok_per_w
        pltpu.sync_copy(pe_hbm, pe_v)

        def load_idx(c, b):
            pltpu.sync_copy(x_hbm.at[pl.ds(tok0 + c * CHUNK, CHUNK)], idx_v.at[b])

        def start_gather(b):
            pltpu.async_copy(table_hbm.at[idx_v.at[b]], rows_v.at[b], gsem[b])

        def wait_gather(b):
            pltpu.make_async_copy(
                table_hbm.at[idx_v.at[b]], rows_v.at[b], gsem[b]
            ).wait()

        def start_scatter(c, b):
            pltpu.async_copy(
                rows_v.at[b], out_hbm.at[pl.ds(tok0 + c * CHUNK, CHUNK)], osem[b]
            )

        def wait_scatter(c, b):
            pltpu.make_async_copy(
                rows_v.at[b], out_hbm.at[pl.ds(tok0 + c * CHUNK, CHUNK)], osem[b]
            ).wait()

        def compute(c, b):
            if isinstance(c, int):
                pe0 = (c * CHUNK) % seq
            else:
                pe0 = lax.rem(c * CHUNK, seq)

            def tok_body(t, carry):
                for j in range(D_MODEL // 16):
                    sl = pl.ds(j * 16, 16)
                    rows_v[b, t, sl] = rows_v[b, t, sl] * SCALE + pe_v[pe0 + t, sl]
                return carry

            lax.fori_loop(0, CHUNK, tok_body, 0, unroll=2)

        def stage(c, b, owait):
            wait_gather(b)
            nb = (b + 2) % NBUF
            load_idx(c + 2, nb)
            if owait:
                wait_scatter(c - 2, nb)
            start_gather(nb)
            compute(c, b)
            start_scatter(c, b)

        load_idx(0, 0)
        start_gather(0)
        load_idx(1, 1)
        start_gather(1)

        for c in range(NBUF):
            stage(c, c % NBUF, owait=(c >= 2))

        def main_body(k, carry):
            c0 = k * NBUF
            for b in range(NBUF):
                stage(c0 + b, b, owait=True)
            return carry

        lax.fori_loop(1, (nchunk - 2) // NBUF, main_body, 0)

        for c in range(nchunk - 2, nchunk):
            b = c % NBUF
            wait_gather(b)
            compute(c, b)
            start_scatter(c, b)

        for c in range(nchunk - NBUF, nchunk):
            wait_scatter(c, c % NBUF)

    return sc_embed


def kernel(x, table):
    batch, seq = x.shape
    x_flat = x.reshape(-1).astype(jnp.int32)
    pe_ext = _pe_extended()
    sc_embed = _make_sc_call(batch, seq)
    out = sc_embed(x_flat, table, pe_ext)
    return out.reshape(batch, seq, D_MODEL)


# speedup vs baseline: 4.3618x; 1.7506x over previous
import functools
import math

import jax
import jax.numpy as jnp
import numpy as np
from jax import lax
from jax.experimental import pallas as pl
from jax.experimental.pallas import tpu as pltpu
from jax.experimental.pallas import tpu_sc as plsc

VOCAB = 1000
D_MODEL = 128
MAX_LEN = 512
SEQ = 200
SCALE = float(math.sqrt(D_MODEL))

CHUNK = 128
NBUF = 4


def _pe_extended() -> jnp.ndarray:
    pe = np.zeros((MAX_LEN, D_MODEL), dtype=np.float32)
    position = np.arange(0, MAX_LEN, dtype=np.float32)[:, None]
    div_term = np.exp(
        np.arange(0, D_MODEL, 2, dtype=np.float32) * (-math.log(10000.0) / D_MODEL)
    )
    pe[:, 0::2] = np.sin(position * div_term)
    pe[:, 1::2] = np.cos(position * div_term)
    pe = pe[:SEQ]
    pe_ext = np.concatenate([pe, pe[:CHUNK]], axis=0)
    return jnp.asarray(pe_ext)


def _make_sc_call(batch: int, seq: int):
    info = plsc.get_sparse_core_info()
    nc, ns = info.num_cores, info.num_subcores
    nw = nc * ns
    ntok = batch * seq
    assert ntok % (nw * CHUNK) == 0
    tok_per_w = ntok // nw
    nchunk = tok_per_w // CHUNK
    pe_rows = seq + CHUNK

    mesh = plsc.VectorSubcoreMesh(core_axis_name="c", subcore_axis_name="s")

    @functools.partial(
        pl.kernel,
        mesh=mesh,
        out_type=jax.ShapeDtypeStruct((ntok, D_MODEL), jnp.float32),
        scratch_types=[
            pltpu.VMEM((NBUF, CHUNK), jnp.int32),
            pltpu.VMEM((NBUF, CHUNK, D_MODEL), jnp.float32),
            pltpu.VMEM((pe_rows, D_MODEL), jnp.float32),
        ]
        + [pltpu.SemaphoreType.DMA] * (2 * NBUF),
    )
    def sc_embed(x_hbm, table_hbm, pe_hbm, out_hbm, idx_v, rows_v, pe_v, *sems):
        gsem = sems[:NBUF]
        osem = sems[NBUF:]
        wid = lax.axis_index("s") * nc + lax.axis_index("c")
        tok0 = wid * tok_per_w
        pltpu.sync_copy(pe_hbm, pe_v)

        def load_idx(c, b):
            pltpu.sync_copy(x_hbm.at[pl.ds(tok0 + c * CHUNK, CHUNK)], idx_v.at[b])

        def start_gather(b):
            pltpu.async_copy(table_hbm.at[idx_v.at[b]], rows_v.at[b], gsem[b])

        def wait_gather(b):
            pltpu.make_async_copy(
                table_hbm.at[idx_v.at[b]], rows_v.at[b], gsem[b]
            ).wait()

        def start_scatter(c, b):
            pltpu.async_copy(
                rows_v.at[b], out_hbm.at[pl.ds(tok0 + c * CHUNK, CHUNK)], osem[b]
            )

        def wait_scatter(c, b):
            pltpu.make_async_copy(
                rows_v.at[b], out_hbm.at[pl.ds(tok0 + c * CHUNK, CHUNK)], osem[b]
            ).wait()

        def compute(c, b):
            if isinstance(c, int):
                pe0 = (c * CHUNK) % seq
            else:
                pe0 = lax.rem(c * CHUNK, seq)

            @plsc.parallel_loop(0, CHUNK, 1, unroll=4)
            def tok_body(t):
                for j in range(D_MODEL // 16):
                    sl = pl.ds(j * 16, 16)
                    rows_v[b, t, sl] = rows_v[b, t, sl] * SCALE + pe_v[pe0 + t, sl]

        def stage(c, b, owait):
            wait_gather(b)
            nb = (b + 2) % NBUF
            load_idx(c + 2, nb)
            if owait:
                wait_scatter(c - 2, nb)
            start_gather(nb)
            compute(c, b)
            start_scatter(c, b)

        load_idx(0, 0)
        start_gather(0)
        load_idx(1, 1)
        start_gather(1)

        for c in range(NBUF):
            stage(c, c % NBUF, owait=(c >= 2))

        def main_body(k, carry):
            c0 = k * NBUF
            for b in range(NBUF):
                stage(c0 + b, b, owait=True)
            return carry

        lax.fori_loop(1, (nchunk - 2) // NBUF, main_body, 0)

        for c in range(nchunk - 2, nchunk):
            b = c % NBUF
            wait_gather(b)
            compute(c, b)
            start_scatter(c, b)

        for c in range(nchunk - NBUF, nchunk):
            wait_scatter(c, c % NBUF)

    return sc_embed


def kernel(x, table):
    batch, seq = x.shape
    x_flat = x.reshape(-1).astype(jnp.int32)
    pe_ext = _pe_extended()
    sc_embed = _make_sc_call(batch, seq)
    out = sc_embed(x_flat, table, pe_ext)
    return out.reshape(batch, seq, D_MODEL)


# speedup vs baseline: 4.3678x; 1.0014x over previous
import functools
import math

import jax
import jax.numpy as jnp
import numpy as np
from jax import lax
from jax.experimental import pallas as pl
from jax.experimental.pallas import tpu as pltpu
from jax.experimental.pallas import tpu_sc as plsc

VOCAB = 1000
D_MODEL = 128
MAX_LEN = 512
SEQ = 200
SCALE = float(math.sqrt(D_MODEL))

CHUNK = 128
NBUF = 4


def _pe_extended() -> jnp.ndarray:
    pe = np.zeros((MAX_LEN, D_MODEL), dtype=np.float32)
    position = np.arange(0, MAX_LEN, dtype=np.float32)[:, None]
    div_term = np.exp(
        np.arange(0, D_MODEL, 2, dtype=np.float32) * (-math.log(10000.0) / D_MODEL)
    )
    pe[:, 0::2] = np.sin(position * div_term)
    pe[:, 1::2] = np.cos(position * div_term)
    pe = pe[:SEQ]
    pe_ext = np.concatenate([pe, pe[:CHUNK]], axis=0)
    return jnp.asarray(pe_ext)


def _make_sc_call(batch: int, seq: int):
    info = plsc.get_sparse_core_info()
    nc, ns = info.num_cores, info.num_subcores
    nw = nc * ns
    ntok = batch * seq
    assert ntok % (nw * CHUNK) == 0
    tok_per_w = ntok // nw
    nchunk = tok_per_w // CHUNK
    pe_rows = seq + CHUNK

    mesh = plsc.VectorSubcoreMesh(core_axis_name="c", subcore_axis_name="s")

    @functools.partial(
        pl.kernel,
        mesh=mesh,
        out_type=jax.ShapeDtypeStruct((ntok, D_MODEL), jnp.float32),
        scratch_types=[
            pltpu.VMEM((tok_per_w,), jnp.int32),
            pltpu.VMEM((NBUF, CHUNK, D_MODEL), jnp.float32),
            pltpu.VMEM((pe_rows, D_MODEL), jnp.float32),
        ]
        + [pltpu.SemaphoreType.DMA] * (2 * NBUF),
    )
    def sc_embed(x_hbm, table_hbm, pe_hbm, out_hbm, idx_v, rows_v, pe_v, *sems):
        gsem = sems[:NBUF]
        osem = sems[NBUF:]
        wid = lax.axis_index("s") * nc + lax.axis_index("c")
        tok0 = wid * tok_per_w
        pltpu.sync_copy(pe_hbm, pe_v)
        pltpu.sync_copy(x_hbm.at[pl.ds(tok0, tok_per_w)], idx_v)

        def start_gather(c, b):
            pltpu.async_copy(
                table_hbm.at[idx_v.at[pl.ds(c * CHUNK, CHUNK)]], rows_v.at[b], gsem[b]
            )

        def wait_gather(c, b):
            pltpu.make_async_copy(
                table_hbm.at[idx_v.at[pl.ds(c * CHUNK, CHUNK)]], rows_v.at[b], gsem[b]
            ).wait()

        def start_scatter(c, b):
            pltpu.async_copy(
                rows_v.at[b], out_hbm.at[pl.ds(tok0 + c * CHUNK, CHUNK)], osem[b]
            )

        def wait_scatter(c, b):
            pltpu.make_async_copy(
                rows_v.at[b], out_hbm.at[pl.ds(tok0 + c * CHUNK, CHUNK)], osem[b]
            ).wait()

        def compute(c, b):
            if isinstance(c, int):
                pe0 = (c * CHUNK) % seq
            else:
                pe0 = lax.rem(c * CHUNK, seq)

            @plsc.parallel_loop(0, CHUNK, 1, unroll=8)
            def tok_body(t):
                for j in range(D_MODEL // 16):
                    sl = pl.ds(j * 16, 16)
                    rows_v[b, t, sl] = rows_v[b, t, sl] * SCALE + pe_v[pe0 + t, sl]

        def stage(c, b, owait):
            wait_gather(c, b)
            nb = (b + 2) % NBUF
            if owait:
                wait_scatter(c - 2, nb)
            start_gather(c + 2, nb)
            compute(c, b)
            start_scatter(c, b)

        start_gather(0, 0)
        start_gather(1, 1)

        for c in range(NBUF):
            stage(c, c % NBUF, owait=(c >= 2))

        def main_body(k, carry):
            c0 = k * NBUF
            for b in range(NBUF):
                stage(c0 + b, b, owait=True)
            return carry

        lax.fori_loop(1, (nchunk - 2) // NBUF, main_body, 0)

        for c in range(nchunk - 2, nchunk):
            b = c % NBUF
            wait_gather(c, b)
            compute(c, b)
            start_scatter(c, b)

        for c in range(nchunk - NBUF, nchunk):
            wait_scatter(c, c % NBUF)

    return sc_embed


def kernel(x, table):
    batch, seq = x.shape
    x_2d = x.reshape(-1).astype(jnp.int32)
    pe_ext = _pe_extended()
    sc_embed = _make_sc_call(batch, seq)
    out = sc_embed(x_2d, table, pe_ext)
    return out.reshape(batch, seq, D_MODEL)


# speedup vs baseline: 6.3643x; 1.4571x over previous
import functools
import math

import jax
import jax.numpy as jnp
import numpy as np
from jax import lax
from jax.experimental import pallas as pl
from jax.experimental.pallas import tpu as pltpu
from jax.experimental.pallas import tpu_sc as plsc

VOCAB = 1000
D_MODEL = 128
MAX_LEN = 512
SEQ = 200
SCALE = float(math.sqrt(D_MODEL))

CHUNK = 128
NBUF = 4
REP = 8


def _pe_extended() -> jnp.ndarray:
    pe = np.zeros((MAX_LEN, D_MODEL), dtype=np.float32)
    position = np.arange(0, MAX_LEN, dtype=np.float32)[:, None]
    div_term = np.exp(
        np.arange(0, D_MODEL, 2, dtype=np.float32) * (-math.log(10000.0) / D_MODEL)
    )
    pe[:, 0::2] = np.sin(position * div_term)
    pe[:, 1::2] = np.cos(position * div_term)
    pe = pe[:SEQ]
    pe_ext = np.concatenate([pe, pe[:CHUNK]], axis=0)
    return jnp.asarray(pe_ext)


def _make_sc_call(batch: int, seq: int):
    info = plsc.get_sparse_core_info()
    nc, ns = info.num_cores, info.num_subcores
    nw = nc * ns
    ntok = batch * seq
    assert ntok % (nw * CHUNK) == 0
    tok_per_w = ntok // nw
    nchunk = tok_per_w // CHUNK
    pe_rows = seq + CHUNK

    mesh = plsc.VectorSubcoreMesh(core_axis_name="c", subcore_axis_name="s")

    @functools.partial(
        pl.kernel,
        mesh=mesh,
        out_type=jax.ShapeDtypeStruct((ntok, D_MODEL), jnp.float32),
        scratch_types=[
            pltpu.VMEM((tok_per_w,), jnp.int32),
            pltpu.VMEM((NBUF, CHUNK, D_MODEL), jnp.float32),
            pltpu.VMEM((pe_rows, D_MODEL), jnp.float32),
        ]
        + [pltpu.SemaphoreType.DMA] * (2 * NBUF),
    )
    def sc_embed(x_hbm, table_hbm, pe_hbm, out_hbm, idx_v, rows_v, pe_v, *sems):
        gsem = sems[:NBUF]
        osem = sems[NBUF:]
        wid = lax.axis_index("s") * nc + lax.axis_index("c")
        tok0 = wid * tok_per_w
        pltpu.sync_copy(pe_hbm, pe_v)
        pltpu.sync_copy(x_hbm.at[pl.ds(tok0, tok_per_w)], idx_v)

        roff = (lax.iota(jnp.int32, 16) % REP) * VOCAB

        @plsc.parallel_loop(0, tok_per_w // 16, 1, unroll=8)
        def idx_adj(v):
            sl = pl.ds(v * 16, 16)
            idx_v[sl] = idx_v[sl] + roff

        def start_gather(c, b):
            pltpu.async_copy(
                table_hbm.at[idx_v.at[pl.ds(c * CHUNK, CHUNK)]], rows_v.at[b], gsem[b]
            )

        def wait_gather(c, b):
            pltpu.make_async_copy(
                table_hbm.at[idx_v.at[pl.ds(c * CHUNK, CHUNK)]], rows_v.at[b], gsem[b]
            ).wait()

        def start_scatter(c, b):
            pltpu.async_copy(
                rows_v.at[b], out_hbm.at[pl.ds(tok0 + c * CHUNK, CHUNK)], osem[b]
            )

        def wait_scatter(c, b):
            pltpu.make_async_copy(
                rows_v.at[b], out_hbm.at[pl.ds(tok0 + c * CHUNK, CHUNK)], osem[b]
            ).wait()

        def compute(c, b):
            if isinstance(c, int):
                pe0 = (c * CHUNK) % seq
            else:
                pe0 = lax.rem(c * CHUNK, seq)

            @plsc.parallel_loop(0, CHUNK, 1, unroll=8)
            def tok_body(t):
                for j in range(D_MODEL // 16):
                    sl = pl.ds(j * 16, 16)
                    rows_v[b, t, sl] = rows_v[b, t, sl] * SCALE + pe_v[pe0 + t, sl]

        def stage(c, b, owait):
            wait_gather(c, b)
            nb = (b + 2) % NBUF
            if owait:
                wait_scatter(c - 2, nb)
            start_gather(c + 2, nb)
            compute(c, b)
            start_scatter(c, b)

        start_gather(0, 0)
        start_gather(1, 1)

        for c in range(NBUF):
            stage(c, c % NBUF, owait=(c >= 2))

        def main_body(k, carry):
            c0 = k * NBUF
            for b in range(NBUF):
                stage(c0 + b, b, owait=True)
            return carry

        lax.fori_loop(1, (nchunk - 2) // NBUF, main_body, 0)

        for c in range(nchunk - 2, nchunk):
            b = c % NBUF
            wait_gather(c, b)
            compute(c, b)
            start_scatter(c, b)

        for c in range(nchunk - NBUF, nchunk):
            wait_scatter(c, c % NBUF)

    return sc_embed


def kernel(x, table):
    batch, seq = x.shape
    x_2d = x.reshape(-1).astype(jnp.int32)
    pe_ext = _pe_extended()
    table_rep = jnp.tile(table, (REP, 1))
    sc_embed = _make_sc_call(batch, seq)
    out = sc_embed(x_2d, table_rep, pe_ext)
    return out.reshape(batch, seq, D_MODEL)


# speedup vs baseline: 6.5074x; 1.0225x over previous
import functools
import math

import jax
import jax.numpy as jnp
import numpy as np
from jax import lax
from jax.experimental import pallas as pl
from jax.experimental.pallas import tpu as pltpu
from jax.experimental.pallas import tpu_sc as plsc

VOCAB = 1000
D_MODEL = 128
MAX_LEN = 512
SEQ = 200
SCALE = float(math.sqrt(D_MODEL))

CHUNK = 128
NBUF = 4
REP = 16


def _pe_extended() -> jnp.ndarray:
    pe = np.zeros((MAX_LEN, D_MODEL), dtype=np.float32)
    position = np.arange(0, MAX_LEN, dtype=np.float32)[:, None]
    div_term = np.exp(
        np.arange(0, D_MODEL, 2, dtype=np.float32) * (-math.log(10000.0) / D_MODEL)
    )
    pe[:, 0::2] = np.sin(position * div_term)
    pe[:, 1::2] = np.cos(position * div_term)
    pe = pe[:SEQ]
    pe_ext = np.concatenate([pe, pe[:CHUNK]], axis=0)
    return jnp.asarray(pe_ext)


def _make_sc_call(batch: int, seq: int):
    info = plsc.get_sparse_core_info()
    nc, ns = info.num_cores, info.num_subcores
    nw = nc * ns
    ntok = batch * seq
    assert ntok % (nw * CHUNK) == 0
    tok_per_w = ntok // nw
    nchunk = tok_per_w // CHUNK
    pe_rows = seq + CHUNK

    mesh = plsc.VectorSubcoreMesh(core_axis_name="c", subcore_axis_name="s")

    @functools.partial(
        pl.kernel,
        mesh=mesh,
        out_type=jax.ShapeDtypeStruct((ntok, D_MODEL), jnp.float32),
        scratch_types=[
            pltpu.VMEM((tok_per_w,), jnp.int32),
            pltpu.VMEM((NBUF, CHUNK, D_MODEL), jnp.float32),
            pltpu.VMEM((pe_rows, D_MODEL), jnp.float32),
        ]
        + [pltpu.SemaphoreType.DMA] * (2 * NBUF),
    )
    def sc_embed(x_hbm, table_hbm, pe_hbm, out_hbm, idx_v, rows_v, pe_v, *sems):
        gsem = sems[:NBUF]
        osem = sems[NBUF:]
        wid = lax.axis_index("s") * nc + lax.axis_index("c")
        tok0 = wid * tok_per_w
        pltpu.sync_copy(pe_hbm, pe_v)
        pltpu.sync_copy(x_hbm.at[pl.ds(tok0, tok_per_w)], idx_v)

        roff = (lax.iota(jnp.int32, 16) % REP) * VOCAB

        @plsc.parallel_loop(0, tok_per_w // 16, 1, unroll=8)
        def idx_adj(v):
            sl = pl.ds(v * 16, 16)
            idx_v[sl] = idx_v[sl] + roff

        def start_gather(c, b):
            pltpu.async_copy(
                table_hbm.at[idx_v.at[pl.ds(c * CHUNK, CHUNK)]], rows_v.at[b], gsem[b]
            )

        def wait_gather(c, b):
            pltpu.make_async_copy(
                table_hbm.at[idx_v.at[pl.ds(c * CHUNK, CHUNK)]], rows_v.at[b], gsem[b]
            ).wait()

        def start_scatter(c, b):
            pltpu.async_copy(
                rows_v.at[b], out_hbm.at[pl.ds(tok0 + c * CHUNK, CHUNK)], osem[b]
            )

        def wait_scatter(c, b):
            pltpu.make_async_copy(
                rows_v.at[b], out_hbm.at[pl.ds(tok0 + c * CHUNK, CHUNK)], osem[b]
            ).wait()

        def compute(c, b):
            if isinstance(c, int):
                pe0 = (c * CHUNK) % seq
            else:
                pe0 = lax.rem(c * CHUNK, seq)

            @plsc.parallel_loop(0, CHUNK, 1, unroll=8)
            def tok_body(t):
                for j in range(D_MODEL // 16):
                    sl = pl.ds(j * 16, 16)
                    rows_v[b, t, sl] = rows_v[b, t, sl] * SCALE + pe_v[pe0 + t, sl]

        def stage(c, b, owait):
            wait_gather(c, b)
            nb = (b + 2) % NBUF
            if owait:
                wait_scatter(c - 2, nb)
            start_gather(c + 2, nb)
            compute(c, b)
            start_scatter(c, b)

        start_gather(0, 0)
        start_gather(1, 1)

        for c in range(NBUF):
            stage(c, c % NBUF, owait=(c >= 2))

        def main_body(k, carry):
            c0 = k * NBUF
            for b in range(NBUF):
                stage(c0 + b, b, owait=True)
            return carry

        lax.fori_loop(1, (nchunk - 2) // NBUF, main_body, 0)

        for c in range(nchunk - 2, nchunk):
            b = c % NBUF
            wait_gather(c, b)
            compute(c, b)
            start_scatter(c, b)

        for c in range(nchunk - NBUF, nchunk):
            wait_scatter(c, c % NBUF)

    return sc_embed


def kernel(x, table):
    batch, seq = x.shape
    x_2d = x.reshape(-1).astype(jnp.int32)
    pe_ext = _pe_extended()
    table_rep = jnp.tile(table, (REP, 1))
    sc_embed = _make_sc_call(batch, seq)
    out = sc_embed(x_2d, table_rep, pe_ext)
    return out.reshape(batch, seq, D_MODEL)
